# Initial kernel scaffold; baseline (speedup 1.0000x reference)
import jax
import jax.numpy as jnp
from jax.experimental import pallas as pl


def kernel(coord, params, at_no, edge_index, charge, spin):
    raise NotImplementedError("write your pallas kernel here")


# speedup vs baseline: 1.8005x; 1.8005x over previous
import functools
import math

import jax
import jax.numpy as jnp
import numpy as np
from jax import lax
from jax.experimental import pallas as pl
from jax.experimental.pallas import tpu as pltpu
from jax.experimental.pallas import tpu_sc as plsc

_N = 10000
_E = 160000
_F = 128
_NB = 20
_CUTOFF = 5.0

_NC = 2
_NS = 16
_NW = _NC * _NS
_PW = _E // _NW
_CH = 40
_NCHUNK = _PW // _CH

@functools.cache
def _mesh():
    return plsc.VectorSubcoreMesh(
        core_axis_name="c", subcore_axis_name="s",
        num_cores=_NC, num_subcores=_NS,
    )


def _int_zero(x):
    return np.zeros(x.shape, dtype=jax.dtypes.float0)


def _pick_bm(m):
    for bm in (1024, 640, 512, 400, 256, 200, 128, 80, 40, 16, 8):
        if m % bm == 0:
            return bm
    return m


def _mm(x, w):
    m, k = x.shape
    b = w.shape[1]
    bm = _pick_bm(m)

    def body(x_ref, w_ref, o_ref):
        o_ref[...] = jnp.dot(x_ref[...], w_ref[...],
                             preferred_element_type=jnp.float32)

    return pl.pallas_call(
        body,
        grid=(m // bm,),
        in_specs=[
            pl.BlockSpec((bm, k), lambda i: (i, 0)),
            pl.BlockSpec((k, b), lambda i: (0, 0)),
        ],
        out_specs=pl.BlockSpec((bm, b), lambda i: (i, 0)),
        out_shape=jax.ShapeDtypeStruct((m, b), jnp.float32),
    )(x, w)


@jax.custom_vjp
def _mm_op(x, w):
    return _mm(x, w)


def _mm_fwd(x, w):
    return _mm(x, w), w


def _mm_bwd(w, g):
    return _mm(g, w.T), jnp.zeros_like(w)


_mm_op.defvjp(_mm_fwd, _mm_bwd)


_RSQRT_F = 1.0 / math.sqrt(_F)


def _attn_k(q, kvec):
    n = q.shape[0]

    def body(q_ref, k_ref, o_ref):
        dot = jnp.sum(q_ref[...] * k_ref[...], axis=1, keepdims=True) * _RSQRT_F
        e = jnp.exp(dot - jnp.max(dot))
        o_ref[...] = e / jnp.sum(e)

    return pl.pallas_call(
        body, out_shape=jax.ShapeDtypeStruct((n, 1), jnp.float32)
    )(q, kvec)


def _attn_bwd_k(p, g, kvec):
    n = p.shape[0]
    f = kvec.shape[1]

    def body(p_ref, g_ref, k_ref, o_ref):
        pv = p_ref[...]
        gv = g_ref[...]
        gd = pv * (gv - jnp.sum(pv * gv))
        o_ref[...] = gd * (k_ref[...] * _RSQRT_F)

    return pl.pallas_call(
        body, out_shape=jax.ShapeDtypeStruct((n, f), jnp.float32)
    )(p, g, kvec)


@jax.custom_vjp
def _attn_op(q, kvec):
    return _attn_k(q, kvec)


def _attn_fwd(q, kvec):
    p = _attn_k(q, kvec)
    return p, (p, kvec)


def _attn_bwd(res, g):
    p, kvec = res
    return _attn_bwd_k(p, g, kvec), jnp.zeros_like(kvec)


_attn_op.defvjp(_attn_fwd, _attn_bwd)


def _sum_k(x):
    def body(x_ref, o_ref):
        o_ref[...] = jnp.sum(x_ref[...], keepdims=True)

    return pl.pallas_call(
        body, out_shape=jax.ShapeDtypeStruct((1, 1), jnp.float32)
    )(x)


@jax.custom_vjp
def _sum_op(x):
    return _sum_k(x)


def _sum_fwd(x):
    return _sum_k(x), None


def _sum_bwd(_, g):
    return (jnp.broadcast_to(g, (1, _N)),)


_sum_op.defvjp(_sum_fwd, _sum_bwd)


def _sc_gather(table, idx):
    c = table.shape[1]
    e = idx.shape[0]
    pw = e // _NW
    nchunk = pw // _CH

    @functools.partial(
        pl.kernel,
        out_type=jax.ShapeDtypeStruct((e, c), jnp.float32),
        mesh=_mesh(),
        scratch_types=[
            pltpu.VMEM((_CH,), jnp.int32),
            pltpu.VMEM((_CH, c), jnp.float32),
            pltpu.SemaphoreType.DMA,
        ],
    )
    def k(table_hbm, idx_hbm, out_hbm, idx_v, rows_v, sem):
        wid = lax.axis_index("s") * _NC + lax.axis_index("c")

        def body(j, carry):
            base = wid * pw + j * _CH
            pltpu.sync_copy(idx_hbm.at[pl.ds(base, _CH)], idx_v)
            pltpu.async_copy(table_hbm.at[idx_v], rows_v, sem).wait()
            pltpu.sync_copy(rows_v, out_hbm.at[pl.ds(base, _CH)])
            return carry

        lax.fori_loop(0, nchunk, body, 0)

    return k(table, idx)


def _sc_scatter_add(vals, idx, n_out):
    c = vals.shape[1]
    n_pad = -(-n_out // (_NS * 8)) * (_NS * 8)
    rpt = n_pad // _NS
    e = vals.shape[0]
    pw = e // _NW
    nchunk = pw // _CH

    @functools.partial(
        pl.kernel,
        out_type=jax.ShapeDtypeStruct((_NC, n_pad, c), jnp.float32),
        mesh=_mesh(),
        scratch_types=[
            pltpu.VMEM((_CH,), jnp.int32),
            pltpu.VMEM((_CH, c), jnp.float32),
            pltpu.VMEM_SHARED((n_pad, c), jnp.float32),
        ],
    )
    def k(vals_hbm, idx_hbm, zeros_hbm, out_hbm, idx_v, vals_v, acc):
        cid = lax.axis_index("c")
        sid = lax.axis_index("s")
        r0 = sid * rpt
        pltpu.sync_copy(zeros_hbm.at[pl.ds(r0, rpt)], acc.at[pl.ds(r0, rpt)])
        plsc.subcore_barrier()
        wid = sid * _NC + cid

        def body(j, carry):
            base = wid * pw + j * _CH
            pltpu.sync_copy(idx_hbm.at[pl.ds(base, _CH)], idx_v)
            pltpu.sync_copy(vals_hbm.at[pl.ds(base, _CH)], vals_v)
            pltpu.sync_copy(vals_v, acc.at[idx_v], add=True)
            return carry

        lax.fori_loop(0, nchunk, body, 0)
        plsc.subcore_barrier()
        pltpu.sync_copy(acc.at[pl.ds(r0, rpt)], out_hbm.at[cid, pl.ds(r0, rpt)])

    out = k(vals, idx, jnp.zeros((n_pad, c), jnp.float32))
    return out[0, :n_out] + out[1, :n_out]


def _scatter_cols(vals, idx, n_out):
    c = vals.shape[1]
    if c <= 192:
        return _sc_scatter_add(vals, idx, n_out)
    parts = [
        _sc_scatter_add(vals[:, c0:c0 + _F], idx, n_out)
        for c0 in range(0, c, _F)
    ]
    return jnp.concatenate(parts, axis=1)


@functools.partial(jax.custom_vjp, nondiff_argnums=(2,))
def _gather_op(table, idx, n_table):
    return _sc_gather(table, idx)


def _gather_fwd(table, idx, n_table):
    return _sc_gather(table, idx), idx


def _gather_bwd(n_table, idx, g):
    return _scatter_cols(g, idx, n_table), _int_zero(idx)


_gather_op.defvjp(_gather_fwd, _gather_bwd)


@functools.partial(jax.custom_vjp, nondiff_argnums=(2,))
def _segsum_op(vals, idx, n_out):
    return _scatter_cols(vals, idx, n_out)


def _segsum_fwd(vals, idx, n_out):
    return _scatter_cols(vals, idx, n_out), idx


def _segsum_bwd(n_out, idx, g):
    return _sc_gather(g, idx), _int_zero(idx)


_segsum_op.defvjp(_segsum_fwd, _segsum_bwd)


def _silu(x):
    return x * jax.nn.sigmoid(x)


def _ele(x, ele_val, wq, bq, wk, bk, wv):
    q = _mm_op(x, wq) + bq
    kvec = ele_val * wk[0] + bk
    vvec = ele_val * wv[0]
    attn = _attn_op(q, kvec[None, :])
    return attn * vvec[None, :]


def kernel(coord, params, at_no, edge_index, charge, spin):
    p = params
    src = edge_index[0].astype(jnp.int32)
    dst = edge_index[1].astype(jnp.int32)
    charge_f = jnp.asarray(charge, jnp.float32).reshape(())
    spin_f = jnp.asarray(spin, jnp.float32).reshape(())
    n = coord.shape[0]

    n_idx_pad = -(-n // (_NW * _CH)) * (_NW * _CH)
    at_pad = jnp.pad(at_no.astype(jnp.int32), (0, n_idx_pad - n))
    x0 = _sc_gather(p['atom_embed'], at_pad)[:n]

    def energy_fn(c):
        cpad = jnp.pad(c, ((0, 0), (0, 125)))
        cd = _gather_op(cpad, dst, n)[:, :3]
        cs = _gather_op(cpad, src, n)[:, :3]
        rel = cd - cs
        dist = jnp.sqrt(jnp.sum(rel * rel, axis=1) + 1e-12)
        rsh = rel / dist[:, None]
        nvec = jnp.arange(1, _NB + 1, dtype=jnp.float32)
        rbf = jnp.sin(nvec[None, :] * jnp.pi * dist[:, None] / _CUTOFF) / dist[:, None]
        dclip = jnp.minimum(dist, _CUTOFF)
        fcut = (0.5 * (jnp.cos(jnp.pi * dclip / _CUTOFF) + 1.0))[:, None]

        x = x0
        xv = jnp.zeros((n, 3, _F), jnp.float32)
        for blk in p['blocks']:
            x = x + _ele(x, charge_f, blk['cqW'], blk['cqb'], blk['ckW'],
                         blk['ckb'], blk['cvW'])
            x = x + _ele(x, spin_f, blk['sqW'], blk['sqb'], blk['skW'],
                         blk['skb'], blk['svW'])
            phi = _mm_op(_silu(_mm_op(x, blk['Wm1']) + blk['bm1']),
                         blk['Wm2']) + blk['bm2']
            wf = (_mm_op(rbf, blk['Wrbf']) + blk['brbf']) * fcut
            phis = _gather_op(phi, src, n)
            m = phis * wf
            f1 = m[:, :_F]
            f2 = m[:, _F:2 * _F]
            f3 = m[:, 2 * _F:]
            xvs = _gather_op(xv.reshape(n, 3 * _F), src, n).reshape(-1, 3, _F)
            ds = _segsum_op(f1, dst, n)
            dv = [
                _segsum_op(rsh[:, d:d + 1] * f2 + xvs[:, d, :] * f3, dst, n)
                for d in range(3)
            ]
            x = x + ds
            xv = xv + jnp.stack(dv, axis=1)
            u = _mm_op(xv.reshape(n * 3, _F), blk['Wu']).reshape(n, 3, _F)
            vv = _mm_op(xv.reshape(n * 3, _F), blk['Wv']).reshape(n, 3, _F)
            vn = jnp.sqrt(jnp.sum(vv * vv, axis=1) + 1e-8)
            a = _mm_op(_silu(_mm_op(jnp.concatenate([x, vn], axis=1),
                                    blk['Wa1']) + blk['ba1']),
                       blk['Wa2']) + blk['ba2']
            a_vv = a[:, :_F]
            a_sv = a[:, _F:2 * _F]
            a_ss = a[:, 2 * _F:]
            xv = xv + a_vv[:, None, :] * u
            x = x + a_sv * jnp.sum(u * vv, axis=1) + a_ss

        atom_out = _mm_op(_silu(_mm_op(x, p['Wo1']) + p['bo1']),
                          p['Wo2']) + p['bo2']
        return _sum_op(atom_out.reshape(1, -1)).reshape(())

    energy, grad = jax.value_and_grad(energy_fn)(coord)
    asp = _sc_gather(jnp.tile(p['atom_sp'][:, None], (1, _F)), at_pad)[:n, :1]
    energy = energy + _sum_op(asp.reshape(1, -1)).reshape(())
    return energy, grad


# speedup vs baseline: 1.9275x; 1.0706x over previous
import functools
import math

import jax
import jax.numpy as jnp
import numpy as np
from jax import lax
from jax.experimental import pallas as pl
from jax.experimental.pallas import tpu as pltpu
from jax.experimental.pallas import tpu_sc as plsc

_N = 10000
_E = 160000
_F = 128
_NB = 20
_CUTOFF = 5.0

_NC = 2
_NS = 16
_NW = _NC * _NS
_PW = _E // _NW
_CH = 40
_NCHUNK = _PW // _CH

@functools.cache
def _mesh():
    return plsc.VectorSubcoreMesh(
        core_axis_name="c", subcore_axis_name="s",
        num_cores=_NC, num_subcores=_NS,
    )


def _int_zero(x):
    return np.zeros(x.shape, dtype=jax.dtypes.float0)


def _pick_bm(m):
    for bm in (1024, 640, 512, 400, 256, 200, 128, 80, 40, 16, 8):
        if m % bm == 0:
            return bm
    return m


def _mm(x, w):
    m, k = x.shape
    b = w.shape[1]
    bm = _pick_bm(m)

    def body(x_ref, w_ref, o_ref):
        o_ref[...] = jnp.dot(x_ref[...], w_ref[...],
                             preferred_element_type=jnp.float32)

    return pl.pallas_call(
        body,
        grid=(m // bm,),
        in_specs=[
            pl.BlockSpec((bm, k), lambda i: (i, 0)),
            pl.BlockSpec((k, b), lambda i: (0, 0)),
        ],
        out_specs=pl.BlockSpec((bm, b), lambda i: (i, 0)),
        out_shape=jax.ShapeDtypeStruct((m, b), jnp.float32),
    )(x, w)


@jax.custom_vjp
def _mm_op(x, w):
    return _mm(x, w)


def _mm_fwd(x, w):
    return _mm(x, w), w


def _mm_bwd(w, g):
    return _mm(g, w.T), jnp.zeros_like(w)


_mm_op.defvjp(_mm_fwd, _mm_bwd)


_RSQRT_F = 1.0 / math.sqrt(_F)


def _attn_k(q, kvec):
    n = q.shape[0]

    def body(q_ref, k_ref, o_ref):
        dot = jnp.sum(q_ref[...] * k_ref[...], axis=1, keepdims=True) * _RSQRT_F
        e = jnp.exp(dot - jnp.max(dot))
        o_ref[...] = e / jnp.sum(e)

    return pl.pallas_call(
        body, out_shape=jax.ShapeDtypeStruct((n, 1), jnp.float32)
    )(q, kvec)


def _attn_bwd_k(p, g, kvec):
    n = p.shape[0]
    f = kvec.shape[1]

    def body(p_ref, g_ref, k_ref, o_ref):
        pv = p_ref[...]
        gv = g_ref[...]
        gd = pv * (gv - jnp.sum(pv * gv))
        o_ref[...] = gd * (k_ref[...] * _RSQRT_F)

    return pl.pallas_call(
        body, out_shape=jax.ShapeDtypeStruct((n, f), jnp.float32)
    )(p, g, kvec)


@jax.custom_vjp
def _attn_op(q, kvec):
    return _attn_k(q, kvec)


def _attn_fwd(q, kvec):
    p = _attn_k(q, kvec)
    return p, (p, kvec)


def _attn_bwd(res, g):
    p, kvec = res
    return _attn_bwd_k(p, g, kvec), jnp.zeros_like(kvec)


_attn_op.defvjp(_attn_fwd, _attn_bwd)


def _sum_k(x):
    def body(x_ref, o_ref):
        o_ref[...] = jnp.sum(x_ref[...], keepdims=True)

    return pl.pallas_call(
        body, out_shape=jax.ShapeDtypeStruct((1, 1), jnp.float32)
    )(x)


@jax.custom_vjp
def _sum_op(x):
    return _sum_k(x)


def _sum_fwd(x):
    return _sum_k(x), None


def _sum_bwd(_, g):
    return (jnp.broadcast_to(g, (1, _N)),)


_sum_op.defvjp(_sum_fwd, _sum_bwd)


def _idx2d(idx, ch):
    e = idx.shape[0]
    pw = e // _NW
    nfull = pw // ch
    rem = pw - nfull * ch
    nrows = nfull + (1 if rem else 0)
    a = idx.astype(jnp.int32).reshape(_NW, pw)
    a = jnp.pad(a, ((0, 0), (0, nrows * ch - pw)))
    return a.reshape(_NW, nrows, ch)


def _sc_gather(table, idx):
    c = table.shape[1]
    e = idx.shape[0]
    ch = 128 if c <= 192 else 64
    pw = e // _NW
    nfull = pw // ch
    rem = pw - nfull * ch
    assert e % _NW == 0 and pw % 8 == 0 and rem % 8 == 0 and nfull >= 2
    nrows = nfull + (1 if rem else 0)
    idx3 = _idx2d(idx, ch)

    @functools.partial(
        pl.kernel,
        out_type=jax.ShapeDtypeStruct((e, c), jnp.float32),
        mesh=_mesh(),
        scratch_types=[
            pltpu.VMEM((nrows, ch), jnp.int32),
            pltpu.VMEM((2, ch, c), jnp.float32),
            pltpu.SemaphoreType.DMA,
            pltpu.SemaphoreType.DMA,
        ],
    )
    def k(table_hbm, idx_hbm, out_hbm, idx_v, buf, gsem, wsem):
        wid = lax.axis_index("s") * _NC + lax.axis_index("c")
        base = wid * pw
        pltpu.sync_copy(idx_hbm.at[wid], idx_v)
        pltpu.async_copy(table_hbm.at[idx_v.at[0]], buf.at[0], gsem)

        def loop(j, carry):
            cb = j % 2
            nb = (j + 1) % 2

            @pl.when(j + 1 < nfull)
            def _():
                @pl.when(j >= 1)
                def _():
                    pltpu.make_async_copy(
                        buf.at[nb], out_hbm.at[pl.ds(base, ch)], wsem).wait()
                pltpu.async_copy(table_hbm.at[idx_v.at[j + 1]], buf.at[nb], gsem)

            pltpu.make_async_copy(
                table_hbm.at[pl.ds(0, ch)], buf.at[cb], gsem).wait()
            pltpu.async_copy(buf.at[cb], out_hbm.at[pl.ds(base + j * ch, ch)], wsem)
            return carry

        lax.fori_loop(0, nfull, loop, 0)
        pltpu.make_async_copy(buf.at[0], out_hbm.at[pl.ds(base, ch)], wsem).wait()
        pltpu.make_async_copy(buf.at[0], out_hbm.at[pl.ds(base, ch)], wsem).wait()
        if rem:
            pltpu.async_copy(table_hbm.at[idx_v.at[nfull]], buf.at[0], gsem).wait()
            pltpu.sync_copy(buf.at[0, pl.ds(0, rem)],
                            out_hbm.at[pl.ds(base + nfull * ch, rem)])

    return k(table, idx3)


def _sc_scatter_add(vals, idx, n_out):
    e, c = vals.shape
    assert c % _F == 0
    g_n = c // _F
    ch = 64
    pw = e // _NW
    nfull = pw // ch
    rem = pw - nfull * ch
    assert e % _NW == 0 and pw % 8 == 0 and rem % 8 == 0 and nfull >= 2
    n_pad = -(-n_out // (_NS * 8)) * (_NS * 8)
    rpt = n_pad // _NS
    idx3 = _idx2d(idx, ch)
    nrows = idx3.shape[1]

    @functools.partial(
        pl.kernel,
        out_type=jax.ShapeDtypeStruct((_NC, n_pad, c), jnp.float32),
        mesh=_mesh(),
        scratch_types=[
            pltpu.VMEM((nrows, ch), jnp.int32),
            pltpu.VMEM((2, ch, _F), jnp.float32),
            pltpu.VMEM((ch, _F), jnp.float32),
            pltpu.VMEM_SHARED((n_pad, _F), jnp.float32),
            pltpu.SemaphoreType.DMA,
            pltpu.SemaphoreType.DMA,
        ],
    )
    def k(vals_hbm, idx_hbm, zeros_hbm, out_hbm,
          idx_v, buf, rem_v, acc, lsem, ssem):
        cid = lax.axis_index("c")
        sid = lax.axis_index("s")
        wid = sid * _NC + cid
        base = wid * pw
        r0 = sid * rpt
        pltpu.sync_copy(idx_hbm.at[wid], idx_v)
        for g in range(g_n):
            c0 = g * _F
            pltpu.sync_copy(zeros_hbm.at[pl.ds(r0, rpt)], acc.at[pl.ds(r0, rpt)])
            if rem:
                pltpu.sync_copy(zeros_hbm.at[pl.ds(0, ch)], rem_v)
            plsc.subcore_barrier()
            pltpu.async_copy(
                vals_hbm.at[pl.ds(base, ch), pl.ds(c0, _F)], buf.at[0], lsem)

            def loop(j, carry):
                cb = j % 2
                nb = (j + 1) % 2

                @pl.when(j + 1 < nfull)
                def _():
                    @pl.when(j >= 1)
                    def _():
                        pltpu.make_async_copy(
                            buf.at[nb], acc.at[pl.ds(0, ch)], ssem).wait()
                    pltpu.async_copy(
                        vals_hbm.at[pl.ds(base + (j + 1) * ch, ch), pl.ds(c0, _F)],
                        buf.at[nb], lsem)

                pltpu.make_async_copy(
                    zeros_hbm.at[pl.ds(0, ch)], buf.at[cb], lsem).wait()
                pltpu.async_copy(buf.at[cb], acc.at[idx_v.at[j]], ssem, add=True)
                return carry

            lax.fori_loop(0, nfull, loop, 0)
            pltpu.make_async_copy(buf.at[0], acc.at[pl.ds(0, ch)], ssem).wait()
            pltpu.make_async_copy(buf.at[0], acc.at[pl.ds(0, ch)], ssem).wait()
            if rem:
                pltpu.sync_copy(
                    vals_hbm.at[pl.ds(base + nfull * ch, rem), pl.ds(c0, _F)],
                    rem_v.at[pl.ds(0, rem)])
                pltpu.async_copy(rem_v, acc.at[idx_v.at[nfull]], ssem,
                                 add=True).wait()
            plsc.subcore_barrier()
            pltpu.sync_copy(acc.at[pl.ds(r0, rpt)],
                            out_hbm.at[cid, pl.ds(r0, rpt), pl.ds(c0, _F)])

    out = k(vals, idx3, jnp.zeros((n_pad, _F), jnp.float32))
    return out[0, :n_out] + out[1, :n_out]


def _scatter_cols(vals, idx, n_out):
    return _sc_scatter_add(vals, idx, n_out)


@functools.partial(jax.custom_vjp, nondiff_argnums=(2,))
def _gather_op(table, idx, n_table):
    return _sc_gather(table, idx)


def _gather_fwd(table, idx, n_table):
    return _sc_gather(table, idx), idx


def _gather_bwd(n_table, idx, g):
    return _scatter_cols(g, idx, n_table), _int_zero(idx)


_gather_op.defvjp(_gather_fwd, _gather_bwd)


@functools.partial(jax.custom_vjp, nondiff_argnums=(2,))
def _segsum_op(vals, idx, n_out):
    return _scatter_cols(vals, idx, n_out)


def _segsum_fwd(vals, idx, n_out):
    return _scatter_cols(vals, idx, n_out), idx


def _segsum_bwd(n_out, idx, g):
    return _sc_gather(g, idx), _int_zero(idx)


_segsum_op.defvjp(_segsum_fwd, _segsum_bwd)


def _silu(x):
    return x * jax.nn.sigmoid(x)


def _ele(x, ele_val, wq, bq, wk, bk, wv):
    q = _mm_op(x, wq) + bq
    kvec = ele_val * wk[0] + bk
    vvec = ele_val * wv[0]
    attn = _attn_op(q, kvec[None, :])
    return attn * vvec[None, :]


def kernel(coord, params, at_no, edge_index, charge, spin):
    p = params
    src = edge_index[0].astype(jnp.int32)
    dst = edge_index[1].astype(jnp.int32)
    charge_f = jnp.asarray(charge, jnp.float32).reshape(())
    spin_f = jnp.asarray(spin, jnp.float32).reshape(())
    n = coord.shape[0]

    n_idx_pad = -(-n // (_NW * _CH)) * (_NW * _CH)
    at_pad = jnp.pad(at_no.astype(jnp.int32), (0, n_idx_pad - n))
    x0 = _sc_gather(p['atom_embed'], at_pad)[:n]

    def energy_fn(c):
        cpad = jnp.pad(c, ((0, 0), (0, 125)))
        both = _gather_op(cpad, jnp.concatenate([dst, src]), n)
        cd = both[:_E, :3]
        cs = both[_E:, :3]
        rel = cd - cs
        dist = jnp.sqrt(jnp.sum(rel * rel, axis=1) + 1e-12)
        rsh = rel / dist[:, None]
        nvec = jnp.arange(1, _NB + 1, dtype=jnp.float32)
        rbf = jnp.sin(nvec[None, :] * jnp.pi * dist[:, None] / _CUTOFF) / dist[:, None]
        dclip = jnp.minimum(dist, _CUTOFF)
        fcut = (0.5 * (jnp.cos(jnp.pi * dclip / _CUTOFF) + 1.0))[:, None]

        x = x0
        xv = jnp.zeros((n, 3, _F), jnp.float32)
        for blk in p['blocks']:
            x = x + _ele(x, charge_f, blk['cqW'], blk['cqb'], blk['ckW'],
                         blk['ckb'], blk['cvW'])
            x = x + _ele(x, spin_f, blk['sqW'], blk['sqb'], blk['skW'],
                         blk['skb'], blk['svW'])
            phi = _mm_op(_silu(_mm_op(x, blk['Wm1']) + blk['bm1']),
                         blk['Wm2']) + blk['bm2']
            wf = (_mm_op(rbf, blk['Wrbf']) + blk['brbf']) * fcut
            ga = _gather_op(
                jnp.concatenate([phi, xv.reshape(n, 3 * _F)], axis=1), src, n)
            phis = ga[:, :3 * _F]
            xvs = ga[:, 3 * _F:].reshape(-1, 3, _F)
            m = phis * wf
            f1 = m[:, :_F]
            f2 = m[:, _F:2 * _F]
            f3 = m[:, 2 * _F:]
            contrib = jnp.concatenate(
                [f1] + [rsh[:, d:d + 1] * f2 + xvs[:, d, :] * f3
                        for d in range(3)], axis=1)
            so = _segsum_op(contrib, dst, n)
            x = x + so[:, :_F]
            xv = xv + so[:, _F:].reshape(n, 3, _F)
            u = _mm_op(xv.reshape(n * 3, _F), blk['Wu']).reshape(n, 3, _F)
            vv = _mm_op(xv.reshape(n * 3, _F), blk['Wv']).reshape(n, 3, _F)
            vn = jnp.sqrt(jnp.sum(vv * vv, axis=1) + 1e-8)
            a = _mm_op(_silu(_mm_op(jnp.concatenate([x, vn], axis=1),
                                    blk['Wa1']) + blk['ba1']),
                       blk['Wa2']) + blk['ba2']
            a_vv = a[:, :_F]
            a_sv = a[:, _F:2 * _F]
            a_ss = a[:, 2 * _F:]
            xv = xv + a_vv[:, None, :] * u
            x = x + a_sv * jnp.sum(u * vv, axis=1) + a_ss

        atom_out = _mm_op(_silu(_mm_op(x, p['Wo1']) + p['bo1']),
                          p['Wo2']) + p['bo2']
        return _sum_op(atom_out.reshape(1, -1)).reshape(())

    energy, grad = jax.value_and_grad(energy_fn)(coord)
    asp = _sc_gather(jnp.tile(p['atom_sp'][:, None], (1, _F)), at_pad)[:n, :1]
    energy = energy + _sum_op(asp.reshape(1, -1)).reshape(())
    return energy, grad


# speedup vs baseline: 2.2144x; 1.1488x over previous
import functools
import math

import jax
import jax.numpy as jnp
import numpy as np
from jax import lax
from jax.experimental import pallas as pl
from jax.experimental.pallas import tpu as pltpu
from jax.experimental.pallas import tpu_sc as plsc

_N = 10000
_E = 160000
_F = 128
_NB = 20
_CUTOFF = 5.0

_NC = 2
_NS = 16
_NW = _NC * _NS
_PW = _E // _NW
_CH = 40
_NCHUNK = _PW // _CH

@functools.cache
def _mesh():
    return plsc.VectorSubcoreMesh(
        core_axis_name="c", subcore_axis_name="s",
        num_cores=_NC, num_subcores=_NS,
    )


def _int_zero(x):
    return np.zeros(x.shape, dtype=jax.dtypes.float0)


def _pick_bm(m):
    for bm in (1024, 640, 512, 400, 256, 200, 128, 80, 40, 16, 8):
        if m % bm == 0:
            return bm
    return m


def _mm(x, w):
    m, k = x.shape
    b = w.shape[1]
    bm = _pick_bm(m)

    def body(x_ref, w_ref, o_ref):
        o_ref[...] = jnp.dot(x_ref[...], w_ref[...],
                             preferred_element_type=jnp.float32)

    return pl.pallas_call(
        body,
        grid=(m // bm,),
        in_specs=[
            pl.BlockSpec((bm, k), lambda i: (i, 0)),
            pl.BlockSpec((k, b), lambda i: (0, 0)),
        ],
        out_specs=pl.BlockSpec((bm, b), lambda i: (i, 0)),
        out_shape=jax.ShapeDtypeStruct((m, b), jnp.float32),
    )(x, w)


def _mm_t(g, w):
    m, k = g.shape
    b = w.shape[0]
    bm = _pick_bm(m)

    def body(g_ref, w_ref, o_ref):
        o_ref[...] = lax.dot_general(
            g_ref[...], w_ref[...],
            dimension_numbers=(((1,), (1,)), ((), ())),
            preferred_element_type=jnp.float32)

    return pl.pallas_call(
        body,
        grid=(m // bm,),
        in_specs=[
            pl.BlockSpec((bm, k), lambda i: (i, 0)),
            pl.BlockSpec((b, k), lambda i: (0, 0)),
        ],
        out_specs=pl.BlockSpec((bm, b), lambda i: (i, 0)),
        out_shape=jax.ShapeDtypeStruct((m, b), jnp.float32),
    )(g, w)


@jax.custom_vjp
def _mm_op(x, w):
    return _mm(x, w)


def _mm_fwd(x, w):
    return _mm(x, w), w


def _mm_bwd(w, g):
    return _mm_t(g, w), jnp.zeros_like(w)


_mm_op.defvjp(_mm_fwd, _mm_bwd)


_SQRT_F = math.sqrt(_F)


def _attn_k(q, kvec):
    n = q.shape[0]

    def body(q_ref, k_ref, p_ref, y_ref):
        y = jnp.sum(q_ref[...] * k_ref[...], axis=1, keepdims=True) / _SQRT_F
        u = jnp.exp(y - jnp.max(y))
        p_ref[...] = u / jnp.sum(u)
        y_ref[...] = y

    return pl.pallas_call(
        body, out_shape=(jax.ShapeDtypeStruct((n, 1), jnp.float32),
                         jax.ShapeDtypeStruct((n, 1), jnp.float32))
    )(q, kvec)


def _attn_bwd_k(y, g, kvec):
    n = y.shape[0]
    f = kvec.shape[1]

    def body(y_ref, g_ref, k_ref, o_ref):
        yv = y_ref[...]
        gv = g_ref[...]
        u = jnp.exp(yv - jnp.max(yv))
        s = jnp.sum(u)
        gy = u * (gv / s - jnp.sum(gv * u) / (s * s))
        o_ref[...] = (gy / _SQRT_F) * k_ref[...]

    return pl.pallas_call(
        body, out_shape=jax.ShapeDtypeStruct((n, f), jnp.float32)
    )(y, g, kvec)


@jax.custom_vjp
def _attn_op(q, kvec):
    return _attn_k(q, kvec)[0]


def _attn_fwd(q, kvec):
    p, y = _attn_k(q, kvec)
    return p, (y, kvec)


def _attn_bwd(res, g):
    y, kvec = res
    return _attn_bwd_k(y, g, kvec), jnp.zeros_like(kvec)


_attn_op.defvjp(_attn_fwd, _attn_bwd)


def _sum_k(x):
    def body(x_ref, o_ref):
        o_ref[...] = jnp.sum(x_ref[...], keepdims=True)

    return pl.pallas_call(
        body, out_shape=jax.ShapeDtypeStruct((1, 1), jnp.float32)
    )(x)


@jax.custom_vjp
def _sum_op(x):
    return _sum_k(x)


def _sum_fwd(x):
    return _sum_k(x), None


def _sum_bwd(_, g):
    return (jnp.broadcast_to(g, (1, _N)),)


_sum_op.defvjp(_sum_fwd, _sum_bwd)


def _idx2d(idx, ch):
    e = idx.shape[0]
    pw = e // _NW
    nfull = pw // ch
    rem = pw - nfull * ch
    nrows = nfull + (1 if rem else 0)
    a = idx.astype(jnp.int32).reshape(_NW, pw)
    a = jnp.pad(a, ((0, 0), (0, nrows * ch - pw)))
    return a.reshape(_NW, nrows, ch)


def _sc_gather(table, idx):
    c = table.shape[1]
    e = idx.shape[0]
    ch = 128 if c <= 192 else 64
    pw = e // _NW
    nfull = pw // ch
    rem = pw - nfull * ch
    assert e % _NW == 0 and pw % 8 == 0 and rem % 8 == 0 and nfull >= 2
    nrows = nfull + (1 if rem else 0)
    idx3 = _idx2d(idx, ch)

    @functools.partial(
        pl.kernel,
        out_type=jax.ShapeDtypeStruct((e, c), jnp.float32),
        mesh=_mesh(),
        scratch_types=[
            pltpu.VMEM((nrows, ch), jnp.int32),
            pltpu.VMEM((2, ch, c), jnp.float32),
            pltpu.SemaphoreType.DMA,
            pltpu.SemaphoreType.DMA,
        ],
    )
    def k(table_hbm, idx_hbm, out_hbm, idx_v, buf, gsem, wsem):
        wid = lax.axis_index("s") * _NC + lax.axis_index("c")
        base = wid * pw
        pltpu.sync_copy(idx_hbm.at[wid], idx_v)
        pltpu.async_copy(table_hbm.at[idx_v.at[0]], buf.at[0], gsem)

        def loop(j, carry):
            cb = j % 2
            nb = (j + 1) % 2

            @pl.when(j + 1 < nfull)
            def _():
                @pl.when(j >= 1)
                def _():
                    pltpu.make_async_copy(
                        buf.at[nb], out_hbm.at[pl.ds(base, ch)], wsem).wait()
                pltpu.async_copy(table_hbm.at[idx_v.at[j + 1]], buf.at[nb], gsem)

            pltpu.make_async_copy(
                table_hbm.at[pl.ds(0, ch)], buf.at[cb], gsem).wait()
            pltpu.async_copy(buf.at[cb], out_hbm.at[pl.ds(base + j * ch, ch)], wsem)
            return carry

        lax.fori_loop(0, nfull, loop, 0)
        pltpu.make_async_copy(buf.at[0], out_hbm.at[pl.ds(base, ch)], wsem).wait()
        pltpu.make_async_copy(buf.at[0], out_hbm.at[pl.ds(base, ch)], wsem).wait()
        if rem:
            pltpu.async_copy(table_hbm.at[idx_v.at[nfull]], buf.at[0], gsem).wait()
            pltpu.sync_copy(buf.at[0, pl.ds(0, rem)],
                            out_hbm.at[pl.ds(base + nfull * ch, rem)])

    return k(table, idx3)


def _sc_scatter_add(vals, idx, n_out):
    e, c = vals.shape
    assert c % _F == 0
    g_n = c // _F
    ch = 64
    pw = e // _NW
    nfull = pw // ch
    rem = pw - nfull * ch
    assert e % _NW == 0 and pw % 8 == 0 and rem % 8 == 0 and nfull >= 2
    n_pad = -(-n_out // (_NS * 8)) * (_NS * 8)
    rpt = n_pad // _NS
    idx3 = _idx2d(idx, ch)
    nrows = idx3.shape[1]

    @functools.partial(
        pl.kernel,
        out_type=jax.ShapeDtypeStruct((_NC, n_pad, c), jnp.float32),
        mesh=_mesh(),
        scratch_types=[
            pltpu.VMEM((nrows, ch), jnp.int32),
            pltpu.VMEM((2, ch, _F), jnp.float32),
            pltpu.VMEM((ch, _F), jnp.float32),
            pltpu.VMEM_SHARED((n_pad, _F), jnp.float32),
            pltpu.SemaphoreType.DMA,
            pltpu.SemaphoreType.DMA,
        ],
    )
    def k(vals_hbm, idx_hbm, zeros_hbm, out_hbm,
          idx_v, buf, rem_v, acc, lsem, ssem):
        cid = lax.axis_index("c")
        sid = lax.axis_index("s")
        wid = sid * _NC + cid
        base = wid * pw
        r0 = sid * rpt
        pltpu.sync_copy(idx_hbm.at[wid], idx_v)
        for g in range(g_n):
            c0 = g * _F
            pltpu.sync_copy(zeros_hbm.at[pl.ds(r0, rpt)], acc.at[pl.ds(r0, rpt)])
            if rem:
                pltpu.sync_copy(zeros_hbm.at[pl.ds(0, ch)], rem_v)
            plsc.subcore_barrier()
            pltpu.async_copy(
                vals_hbm.at[pl.ds(base, ch), pl.ds(c0, _F)], buf.at[0], lsem)

            def loop(j, carry):
                cb = j % 2
                nb = (j + 1) % 2

                @pl.when(j + 1 < nfull)
                def _():
                    @pl.when(j >= 1)
                    def _():
                        pltpu.make_async_copy(
                            buf.at[nb], acc.at[pl.ds(0, ch)], ssem).wait()
                    pltpu.async_copy(
                        vals_hbm.at[pl.ds(base + (j + 1) * ch, ch), pl.ds(c0, _F)],
                        buf.at[nb], lsem)

                pltpu.make_async_copy(
                    zeros_hbm.at[pl.ds(0, ch)], buf.at[cb], lsem).wait()
                pltpu.async_copy(buf.at[cb], acc.at[idx_v.at[j]], ssem, add=True)
                return carry

            lax.fori_loop(0, nfull, loop, 0)
            pltpu.make_async_copy(buf.at[0], acc.at[pl.ds(0, ch)], ssem).wait()
            pltpu.make_async_copy(buf.at[0], acc.at[pl.ds(0, ch)], ssem).wait()
            if rem:
                pltpu.sync_copy(
                    vals_hbm.at[pl.ds(base + nfull * ch, rem), pl.ds(c0, _F)],
                    rem_v.at[pl.ds(0, rem)])
                pltpu.async_copy(rem_v, acc.at[idx_v.at[nfull]], ssem,
                                 add=True).wait()
            plsc.subcore_barrier()
            pltpu.sync_copy(acc.at[pl.ds(r0, rpt)],
                            out_hbm.at[cid, pl.ds(r0, rpt), pl.ds(c0, _F)])

    out = k(vals, idx3, jnp.zeros((n_pad, _F), jnp.float32))
    return out[0, :n_out] + out[1, :n_out]


def _scatter_cols(vals, idx, n_out):
    return _sc_scatter_add(vals, idx, n_out)


@functools.partial(jax.custom_vjp, nondiff_argnums=(2,))
def _gather_op(table, idx, n_table):
    return _sc_gather(table, idx)


def _gather_fwd(table, idx, n_table):
    return _sc_gather(table, idx), idx


def _gather_bwd(n_table, idx, g):
    return _scatter_cols(g, idx, n_table), _int_zero(idx)


_gather_op.defvjp(_gather_fwd, _gather_bwd)


@functools.partial(jax.custom_vjp, nondiff_argnums=(2,))
def _segsum_op(vals, idx, n_out):
    return _scatter_cols(vals, idx, n_out)


def _segsum_fwd(vals, idx, n_out):
    return _scatter_cols(vals, idx, n_out), idx


def _segsum_bwd(n_out, idx, g):
    return _sc_gather(g, idx), _int_zero(idx)


_segsum_op.defvjp(_segsum_fwd, _segsum_bwd)


def _silu(x):
    return x * jax.nn.sigmoid(x)


def _ele(x, ele_val, wq, bq, wk, bk, wv):
    q = _mm_op(x, wq) + bq
    kvec = ele_val * wk[0] + bk
    vvec = ele_val * wv[0]
    attn = _attn_op(q, kvec[None, :])
    return attn * vvec[None, :]


def kernel(coord, params, at_no, edge_index, charge, spin):
    p = params
    src = edge_index[0].astype(jnp.int32)
    dst = edge_index[1].astype(jnp.int32)
    charge_f = jnp.asarray(charge, jnp.float32).reshape(())
    spin_f = jnp.asarray(spin, jnp.float32).reshape(())
    n = coord.shape[0]

    n_idx_pad = -(-n // (_NW * _CH)) * (_NW * _CH)
    at_pad = jnp.pad(at_no.astype(jnp.int32), (0, n_idx_pad - n))
    x0 = _sc_gather(p['atom_embed'], at_pad)[:n]

    def energy_fn(c):
        cpad = jnp.pad(c, ((0, 0), (0, 125)))
        both = _gather_op(cpad, jnp.concatenate([dst, src]), n)
        cd = both[:_E, :3]
        cs = both[_E:, :3]
        rel = cd - cs
        dist = jnp.sqrt(jnp.sum(rel * rel, axis=1) + 1e-12)
        rsh = rel / dist[:, None]
        nvec = jnp.arange(1, _NB + 1, dtype=jnp.float32)
        rbf = jnp.sin(nvec[None, :] * jnp.pi * dist[:, None] / _CUTOFF) / dist[:, None]
        dclip = jnp.minimum(dist, _CUTOFF)
        fcut = (0.5 * (jnp.cos(jnp.pi * dclip / _CUTOFF) + 1.0))[:, None]

        x = x0
        xv = jnp.zeros((n, 3, _F), jnp.float32)
        for bi, blk in enumerate(p['blocks']):
            x = x + _ele(x, charge_f, blk['cqW'], blk['cqb'], blk['ckW'],
                         blk['ckb'], blk['cvW'])
            x = x + _ele(x, spin_f, blk['sqW'], blk['sqb'], blk['skW'],
                         blk['skb'], blk['svW'])
            h = _silu(_mm_op(x, blk['Wm1']) + blk['bm1'])
            if bi == 0:
                phi = _mm_op(h, blk['Wm2'][:, :2 * _F]) + blk['bm2'][:2 * _F]
                wf = (_mm_op(rbf, blk['Wrbf'][:, :2 * _F])
                      + blk['brbf'][:2 * _F]) * fcut
                phis = _gather_op(phi, src, n)
                m = phis * wf
                f1 = m[:, :_F]
                f2 = m[:, _F:]
                contrib = jnp.concatenate(
                    [f1] + [rsh[:, d:d + 1] * f2 for d in range(3)], axis=1)
            else:
                phi = _mm_op(h, blk['Wm2']) + blk['bm2']
                wf = (_mm_op(rbf, blk['Wrbf']) + blk['brbf']) * fcut
                ga = _gather_op(
                    jnp.concatenate([phi, xv.reshape(n, 3 * _F)], axis=1),
                    src, n)
                phis = ga[:, :3 * _F]
                xvs = ga[:, 3 * _F:].reshape(-1, 3, _F)
                m = phis * wf
                f1 = m[:, :_F]
                f2 = m[:, _F:2 * _F]
                f3 = m[:, 2 * _F:]
                contrib = jnp.concatenate(
                    [f1] + [rsh[:, d:d + 1] * f2 + xvs[:, d, :] * f3
                            for d in range(3)], axis=1)
            so = _segsum_op(contrib, dst, n)
            x = x + so[:, :_F]
            xv = xv + so[:, _F:].reshape(n, 3, _F)
            u = _mm_op(xv.reshape(n * 3, _F), blk['Wu']).reshape(n, 3, _F)
            vv = _mm_op(xv.reshape(n * 3, _F), blk['Wv']).reshape(n, 3, _F)
            vn = jnp.sqrt(jnp.sum(vv * vv, axis=1) + 1e-8)
            a = _mm_op(_silu(_mm_op(jnp.concatenate([x, vn], axis=1),
                                    blk['Wa1']) + blk['ba1']),
                       blk['Wa2']) + blk['ba2']
            a_vv = a[:, :_F]
            a_sv = a[:, _F:2 * _F]
            a_ss = a[:, 2 * _F:]
            xv = xv + a_vv[:, None, :] * u
            x = x + a_sv * jnp.sum(u * vv, axis=1) + a_ss

        atom_out = _mm_op(_silu(_mm_op(x, p['Wo1']) + p['bo1']),
                          p['Wo2']) + p['bo2']
        return _sum_op(atom_out.reshape(1, -1)).reshape(())

    energy, grad = jax.value_and_grad(energy_fn)(coord)
    asp = _sc_gather(jnp.tile(p['atom_sp'][:, None], (1, _F)), at_pad)[:n, :1]
    energy = energy + _sum_op(asp.reshape(1, -1)).reshape(())
    return energy, grad
